# baseline (speedup 1.0000x reference)
import functools
import math

import jax
import jax.numpy as jnp
from jax import lax
from jax.experimental import pallas as pl
from jax.experimental.pallas import tpu as pltpu
from jax.experimental.pallas import tpu_sc as plsc

_TOPK = 32
_NW = 32
_L = 16
_RB = 32
_CAND = 1600 + 4 * _L


def _sdesc(x):
    return plsc.sort_key_val(x, x, descending=True)[0]


def _sasc(x):
    return plsc.sort_key_val(x, x, descending=False)[0]


def _min_scalar(x):
    return plsc.sort_key_val(x, x, descending=False)[0][0]


def _merge16(a0, a1, v):
    x = _sasc(jnp.maximum(a1, _sasc(v)))
    hi = jnp.maximum(a0, x)
    lo = jnp.minimum(a0, x)
    return _sdesc(hi), _sdesc(lo)


def _sc_body(sim_hbm, thr_hbm, buf, cand, th, sem, *, t_dim, n_batches):
    wid = lax.axis_index("s") * 2 + lax.axis_index("c")
    nvec = t_dim // _L
    n_local = (n_batches - 1 - wid) // _NW + 1

    def dma(bi, slot):
        return pltpu.async_copy(
            sim_hbm.at[pl.ds((wid + bi * _NW) * _RB, _RB)],
            buf.at[pl.ds(slot * _RB, _RB)], sem)

    dma(0, 0)

    def batch_body(bi, carry):
        slot = lax.rem(bi, 2)
        row0 = (wid + bi * _NW) * _RB
        pltpu.make_async_copy(
            sim_hbm.at[pl.ds(row0, _RB)],
            buf.at[pl.ds(slot * _RB, _RB)], sem).wait()

        @pl.when(bi + 1 < n_local)
        def _():
            dma(bi + 1, lax.rem(bi + 1, 2))

        def row_pair_body(ip, carry2):
            _one_row(ip * 2, 0)
            _one_row(ip * 2 + 1, _CAND)
            return carry2

        def _one_row(i, cb):
            rb = slot * _RB + i
            accs = [buf[rb, pl.ds(k * _L, _L)] for k in range(8)]
            for k in range(8, nvec):
                accs[k % 8] = jnp.maximum(accs[k % 8],
                                          buf[rb, pl.ds(k * _L, _L)])
            g0 = jnp.maximum(jnp.maximum(accs[0], accs[1]),
                             jnp.maximum(accs[2], accs[3]))
            g1 = jnp.maximum(jnp.maximum(accs[4], accs[5]),
                             jnp.maximum(accs[6], accs[7]))
            tlo_v = jnp.broadcast_to(_min_scalar(jnp.minimum(g0, g1)), (_L,))
            chain = nvec // 4 + 1
            offs = [0, 0, 0, 0]
            for k in range(nvec):
                j = k % 4
                v = buf[rb, pl.ds(k * _L, _L)]
                msk = v >= tlo_v
                plsc.store_compressed(
                    cand.at[pl.ds(cb + j * chain * _L + offs[j], _L)],
                    v, mask=msk)
                c = plsc.all_reduce_population_count(msk)
                offs[j] = offs[j] + c[0]
            neg = jnp.broadcast_to(jnp.float32(-jnp.inf), (_L,))
            for j in range(4):
                cand[pl.ds(cb + j * chain * _L + offs[j], _L)] = neg
            a0, a1 = neg, neg

            for j in range(4):
                def mbody(k, ac, _j=j):
                    return _merge16(
                        *ac, cand[pl.ds(cb + _j * chain * _L + k * _L, _L)])

                nv = (offs[j] + _L - 1) // _L
                a0, a1 = lax.fori_loop(0, nv, mbody, (a0, a1))
            theta = a1[_L - 1]
            th[i, :] = jnp.broadcast_to(theta, (_L,))

        lax.fori_loop(0, _RB // 2, row_pair_body, 0)
        pltpu.sync_copy(th, thr_hbm.at[pl.ds(row0, _RB)])
        return carry

    lax.fori_loop(0, n_local, batch_body, 0)


def _a_body(tg_ref, tfb_ref, sim_ref, *, d):
    sim_ref[...] = jax.lax.dot_general(
        tg_ref[...], tfb_ref[...], (((1,), (1,)), ((), ())),
        preferred_element_type=jnp.float32,
    ) * (1.0 / math.sqrt(d))


def _b_body(scale_ref, sim_ref, thr_ref, tfe_ref, out_ref, attn_ref):
    sim = sim_ref[...]
    t = thr_ref[...][:, 0:1]
    m = jnp.max(sim, axis=-1, keepdims=True)
    e = jnp.exp(sim - m)
    z = jnp.sum(e, axis=-1, keepdims=True)
    masked = jnp.where(sim >= t, e, 0.0)
    s = jnp.sum(masked, axis=-1, keepdims=True)
    attn = masked * (1.0 / (s + 1e-8 * z))
    attn_ref[...] = attn
    out = jax.lax.dot_general(
        tfe_ref[...], attn.astype(jnp.bfloat16), (((1,), (1,)), ((), ())),
        preferred_element_type=jnp.float32,
    )
    out_ref[...] = scale_ref[0, 0] * out


def kernel(tg_dec, tf_base, tf_expr, scale):
    g, d = tg_dec.shape
    t_dim = tf_base.shape[0]
    p = tf_expr.shape[0]
    r = 512
    grid = ((g + r - 1) // r,)
    n_batches = g // _RB
    scale2 = jnp.asarray(scale, jnp.float32).reshape(1, 1)

    sim = pl.pallas_call(
        functools.partial(_a_body, d=d),
        grid=grid,
        in_specs=[
            pl.BlockSpec((r, d), lambda i: (i, 0)),
            pl.BlockSpec((t_dim, d), lambda i: (0, 0)),
        ],
        out_specs=pl.BlockSpec((r, t_dim), lambda i: (i, 0)),
        out_shape=jax.ShapeDtypeStruct((g, t_dim), jnp.float32),
        compiler_params=pltpu.CompilerParams(
            dimension_semantics=("parallel",),
        ),
    )(tg_dec, tf_base)

    mesh = plsc.VectorSubcoreMesh(core_axis_name="c", subcore_axis_name="s")
    thr = pl.kernel(
        functools.partial(_sc_body, t_dim=t_dim, n_batches=n_batches),
        out_type=jax.ShapeDtypeStruct((g, _L), jnp.float32),
        mesh=mesh,
        compiler_params=pltpu.CompilerParams(needs_layout_passes=False),
        scratch_types=[
            pltpu.VMEM((2 * _RB, t_dim), jnp.float32),
            pltpu.VMEM((2 * _CAND,), jnp.float32),
            pltpu.VMEM((_RB, _L), jnp.float32),
            pltpu.SemaphoreType.DMA,
        ],
    )(sim)

    tf_scalar, attn = pl.pallas_call(
        _b_body,
        grid=grid,
        in_specs=[
            pl.BlockSpec((1, 1), lambda i: (0, 0), memory_space=pltpu.SMEM),
            pl.BlockSpec((r, t_dim), lambda i: (i, 0)),
            pl.BlockSpec((r, _L), lambda i: (i, 0)),
            pl.BlockSpec((p, t_dim), lambda i: (0, 0)),
        ],
        out_specs=(
            pl.BlockSpec((p, r), lambda i: (0, i)),
            pl.BlockSpec((r, t_dim), lambda i: (i, 0)),
        ),
        out_shape=(
            jax.ShapeDtypeStruct((p, g), jnp.float32),
            jax.ShapeDtypeStruct((g, t_dim), jnp.float32),
        ),
        compiler_params=pltpu.CompilerParams(
            dimension_semantics=("parallel",),
        ),
    )(scale2, sim, thr, tf_expr.astype(jnp.bfloat16))
    return tf_scalar, attn


# speedup vs baseline: 1.2202x; 1.2202x over previous
import functools
import math

import jax
import jax.numpy as jnp
from jax import lax
from jax.experimental import pallas as pl
from jax.experimental.pallas import tpu as pltpu
from jax.experimental.pallas import tpu_sc as plsc

_TOPK = 32
_NW = 32
_L = 16
_RB = 32
_CAND = 1600 + 4 * _L


def _sdesc(x):
    return plsc.sort_key_val(x, x, descending=True)[0]


def _sasc(x):
    return plsc.sort_key_val(x, x, descending=False)[0]


def _min_scalar(x):
    return plsc.sort_key_val(x, x, descending=False)[0][0]


def _merge16(a0, a1, v):
    x = _sasc(jnp.maximum(a1, _sasc(v)))
    hi = jnp.maximum(a0, x)
    lo = jnp.minimum(a0, x)
    return _sdesc(hi), _sdesc(lo)


def _sc_body(sim_hbm, thr_hbm, buf, cand, th, sem, *, t_dim, j0, n_batches):
    wid = lax.axis_index("s") * 2 + lax.axis_index("c")
    nvec = t_dim // _L
    n_local = (n_batches - 1 - wid) // _NW + 1

    def dma(bi, slot):
        return pltpu.async_copy(
            sim_hbm.at[pl.ds((j0 + wid + bi * _NW) * _RB, _RB)],
            buf.at[pl.ds(slot * _RB, _RB)], sem)

    dma(0, 0)

    def batch_body(bi, carry):
        slot = lax.rem(bi, 2)
        row0 = (j0 + wid + bi * _NW) * _RB
        pltpu.make_async_copy(
            sim_hbm.at[pl.ds(row0, _RB)],
            buf.at[pl.ds(slot * _RB, _RB)], sem).wait()

        @pl.when(bi + 1 < n_local)
        def _():
            dma(bi + 1, lax.rem(bi + 1, 2))

        def row_body(i, carry2):
            cb = 0
            rb = slot * _RB + i
            accs = [buf[rb, pl.ds(k * _L, _L)] for k in range(8)]
            for k in range(8, nvec):
                accs[k % 8] = jnp.maximum(accs[k % 8],
                                          buf[rb, pl.ds(k * _L, _L)])
            g0 = jnp.maximum(jnp.maximum(accs[0], accs[1]),
                             jnp.maximum(accs[2], accs[3]))
            g1 = jnp.maximum(jnp.maximum(accs[4], accs[5]),
                             jnp.maximum(accs[6], accs[7]))
            tlo_v = jnp.broadcast_to(_min_scalar(jnp.minimum(g0, g1)), (_L,))
            chain = nvec // 4 + 1
            offs = [0, 0, 0, 0]
            for k in range(nvec):
                j = k % 4
                v = buf[rb, pl.ds(k * _L, _L)]
                msk = v >= tlo_v
                plsc.store_compressed(
                    cand.at[pl.ds(cb + j * chain * _L + offs[j], _L)],
                    v, mask=msk)
                c = plsc.all_reduce_population_count(msk)
                offs[j] = offs[j] + c[0]
            neg = jnp.broadcast_to(jnp.float32(-jnp.inf), (_L,))
            for j in range(4):
                cand[pl.ds(cb + j * chain * _L + offs[j], _L)] = neg
            a0, a1 = neg, neg

            for j in range(4):
                def mbody(k, ac, _j=j):
                    return _merge16(
                        *ac, cand[pl.ds(cb + _j * chain * _L + k * _L, _L)])

                nv = (offs[j] + _L - 1) // _L
                a0, a1 = lax.fori_loop(0, nv, mbody, (a0, a1))
            theta = a1[_L - 1]
            th[i, :] = jnp.broadcast_to(theta, (_L,))
            return carry2

        lax.fori_loop(0, _RB, row_body, 0)
        pltpu.sync_copy(th, thr_hbm.at[pl.ds(row0, _RB)])
        return carry

    lax.fori_loop(0, n_local, batch_body, 0)


def _a_body(tg_ref, tfb_ref, sim_ref, *, d):
    sim_ref[...] = jax.lax.dot_general(
        tg_ref[...], tfb_ref[...], (((1,), (1,)), ((), ())),
        preferred_element_type=jnp.float32,
    ) * (1.0 / math.sqrt(d))


def _b1_body(scale_ref, sim_ref, tfe_ref, out_ref, attn_ref):
    sim = sim_ref[...]
    m = jnp.max(sim, axis=-1, keepdims=True)
    e = jnp.exp(sim - m)
    z = jnp.sum(e, axis=-1, keepdims=True)

    def step(_, t):
        return jnp.max(jnp.where(e < t, e, -1.0), axis=-1, keepdims=True)

    t = lax.fori_loop(0, _TOPK, step, jnp.full_like(z, jnp.inf))
    masked = jnp.where(e >= t, e, 0.0)
    s = jnp.sum(masked, axis=-1, keepdims=True)
    attn = masked * (1.0 / (s + 1e-8 * z))
    attn_ref[...] = attn
    out = jax.lax.dot_general(
        tfe_ref[...], attn.astype(jnp.bfloat16), (((1,), (1,)), ((), ())),
        preferred_element_type=jnp.float32,
    )
    out_ref[...] = scale_ref[0, 0] * out


def _b_body(scale_ref, sim_ref, thr_ref, tfe_ref, out_ref, attn_ref):
    sim = sim_ref[...]
    t = thr_ref[...][:, 0:1]
    m = jnp.max(sim, axis=-1, keepdims=True)
    e = jnp.exp(sim - m)
    z = jnp.sum(e, axis=-1, keepdims=True)
    masked = jnp.where(sim >= t, e, 0.0)
    s = jnp.sum(masked, axis=-1, keepdims=True)
    attn = masked * (1.0 / (s + 1e-8 * z))
    attn_ref[...] = attn
    out = jax.lax.dot_general(
        tfe_ref[...], attn.astype(jnp.bfloat16), (((1,), (1,)), ((), ())),
        preferred_element_type=jnp.float32,
    )
    out_ref[...] = scale_ref[0, 0] * out


def kernel(tg_dec, tf_base, tf_expr, scale):
    g, d = tg_dec.shape
    t_dim = tf_base.shape[0]
    p = tf_expr.shape[0]
    r = 512
    grid = ((g + r - 1) // r,)
    nb1 = 24
    g1 = nb1 * r
    g2 = g - g1
    nb2 = (g2 + r - 1) // r
    scale2 = jnp.asarray(scale, jnp.float32).reshape(1, 1)
    tfe_bf = tf_expr.astype(jnp.bfloat16)

    sim = pl.pallas_call(
        functools.partial(_a_body, d=d),
        grid=grid,
        in_specs=[
            pl.BlockSpec((r, d), lambda i: (i, 0)),
            pl.BlockSpec((t_dim, d), lambda i: (0, 0)),
        ],
        out_specs=pl.BlockSpec((r, t_dim), lambda i: (i, 0)),
        out_shape=jax.ShapeDtypeStruct((g, t_dim), jnp.float32),
        compiler_params=pltpu.CompilerParams(
            dimension_semantics=("parallel",),
        ),
    )(tg_dec, tf_base)

    mesh = plsc.VectorSubcoreMesh(core_axis_name="c", subcore_axis_name="s")
    thr = pl.kernel(
        functools.partial(_sc_body, t_dim=t_dim, j0=g1 // _RB,
                          n_batches=(g - g1) // _RB),
        out_type=jax.ShapeDtypeStruct((g, _L), jnp.float32),
        mesh=mesh,
        compiler_params=pltpu.CompilerParams(needs_layout_passes=False),
        scratch_types=[
            pltpu.VMEM((2 * _RB, t_dim), jnp.float32),
            pltpu.VMEM((2 * _CAND,), jnp.float32),
            pltpu.VMEM((_RB, _L), jnp.float32),
            pltpu.SemaphoreType.DMA,
        ],
    )(sim)

    out1, attn1 = pl.pallas_call(
        _b1_body,
        grid=(nb1,),
        in_specs=[
            pl.BlockSpec((1, 1), lambda i: (0, 0), memory_space=pltpu.SMEM),
            pl.BlockSpec((r, t_dim), lambda i: (i, 0)),
            pl.BlockSpec((p, t_dim), lambda i: (0, 0)),
        ],
        out_specs=(
            pl.BlockSpec((p, r), lambda i: (0, i)),
            pl.BlockSpec((r, t_dim), lambda i: (i, 0)),
        ),
        out_shape=(
            jax.ShapeDtypeStruct((p, g1), jnp.float32),
            jax.ShapeDtypeStruct((g1, t_dim), jnp.float32),
        ),
        compiler_params=pltpu.CompilerParams(
            dimension_semantics=("parallel",),
        ),
    )(scale2, sim, tfe_bf)

    out2, attn2 = pl.pallas_call(
        _b_body,
        grid=(nb2,),
        in_specs=[
            pl.BlockSpec((1, 1), lambda i: (0, 0), memory_space=pltpu.SMEM),
            pl.BlockSpec((r, t_dim), lambda i: (i + nb1, 0)),
            pl.BlockSpec((r, _L), lambda i: (i + nb1, 0)),
            pl.BlockSpec((p, t_dim), lambda i: (0, 0)),
        ],
        out_specs=(
            pl.BlockSpec((p, r), lambda i: (0, i)),
            pl.BlockSpec((r, t_dim), lambda i: (i, 0)),
        ),
        out_shape=(
            jax.ShapeDtypeStruct((p, g2), jnp.float32),
            jax.ShapeDtypeStruct((g2, t_dim), jnp.float32),
        ),
        compiler_params=pltpu.CompilerParams(
            dimension_semantics=("parallel",),
        ),
    )(scale2, sim, thr, tfe_bf)
    tf_scalar = jnp.concatenate([out1, out2], axis=1)
    attn = jnp.concatenate([attn1, attn2], axis=0)
    return tf_scalar, attn


# speedup vs baseline: 1.3721x; 1.1245x over previous
import functools
import math

import jax
import jax.numpy as jnp
from jax import lax
from jax.experimental import pallas as pl
from jax.experimental.pallas import tpu as pltpu
from jax.experimental.pallas import tpu_sc as plsc

_TOPK = 32
_NW = 32
_L = 16
_RB = 32
_CAND = 1600 + 4 * _L


def _sdesc(x):
    return plsc.sort_key_val(x, x, descending=True)[0]


def _sasc(x):
    return plsc.sort_key_val(x, x, descending=False)[0]


def _min_scalar(x):
    return plsc.sort_key_val(x, x, descending=False)[0][0]


def _merge16(a0, a1, v):
    x = _sasc(jnp.maximum(a1, _sasc(v)))
    hi = jnp.maximum(a0, x)
    lo = jnp.minimum(a0, x)
    return _sdesc(hi), _sdesc(lo)


def _sc_body(sim_hbm, thr_hbm, buf, cand, th, sem, *, t_dim, j0, n_batches):
    wid = lax.axis_index("s") * 2 + lax.axis_index("c")
    nvec = t_dim // _L
    n_local = (n_batches - 1 - wid) // _NW + 1

    def dma(bi, slot):
        return pltpu.async_copy(
            sim_hbm.at[pl.ds((j0 + wid + bi * _NW) * _RB, _RB)],
            buf.at[pl.ds(slot * _RB, _RB)], sem)

    dma(0, 0)

    def batch_body(bi, carry):
        slot = lax.rem(bi, 2)
        row0 = (j0 + wid + bi * _NW) * _RB
        pltpu.make_async_copy(
            sim_hbm.at[pl.ds(row0, _RB)],
            buf.at[pl.ds(slot * _RB, _RB)], sem).wait()

        @pl.when(bi + 1 < n_local)
        def _():
            dma(bi + 1, lax.rem(bi + 1, 2))

        def row_body(i, carry2):
            cb = 0
            rb = slot * _RB + i
            accs = [buf[rb, pl.ds(k * _L, _L)] for k in range(8)]
            for k in range(8, nvec):
                accs[k % 8] = jnp.maximum(accs[k % 8],
                                          buf[rb, pl.ds(k * _L, _L)])
            g0 = jnp.maximum(jnp.maximum(accs[0], accs[1]),
                             jnp.maximum(accs[2], accs[3]))
            g1 = jnp.maximum(jnp.maximum(accs[4], accs[5]),
                             jnp.maximum(accs[6], accs[7]))
            tlo_v = jnp.broadcast_to(_min_scalar(jnp.minimum(g0, g1)), (_L,))
            chain = nvec // 4 + 1
            offs = [0, 0, 0, 0]
            for k in range(nvec):
                j = k % 4
                v = buf[rb, pl.ds(k * _L, _L)]
                msk = v >= tlo_v
                plsc.store_compressed(
                    cand.at[pl.ds(cb + j * chain * _L + offs[j], _L)],
                    v, mask=msk)
                c = plsc.all_reduce_population_count(msk)
                offs[j] = offs[j] + c[0]
            neg = jnp.broadcast_to(jnp.float32(-jnp.inf), (_L,))
            for j in range(4):
                cand[pl.ds(cb + j * chain * _L + offs[j], _L)] = neg
            a0, a1 = neg, neg

            for j in range(4):
                def mbody(k, ac, _j=j):
                    return _merge16(
                        *ac, cand[pl.ds(cb + _j * chain * _L + k * _L, _L)])

                nv = (offs[j] + _L - 1) // _L
                a0, a1 = lax.fori_loop(0, nv, mbody, (a0, a1))
            theta = a1[_L - 1]
            th[i, :] = jnp.broadcast_to(theta, (_L,))
            return carry2

        lax.fori_loop(0, _RB, row_body, 0)
        pltpu.sync_copy(th, thr_hbm.at[pl.ds(row0, _RB)])
        return carry

    lax.fori_loop(0, n_local, batch_body, 0)


def _a_body(tg_ref, tfb_ref, sim_ref, *, d):
    sim_ref[...] = jax.lax.dot_general(
        tg_ref[...], tfb_ref[...], (((1,), (1,)), ((), ())),
        preferred_element_type=jnp.float32,
    ) * (1.0 / math.sqrt(d))


def _b1_body(scale_ref, sim_ref, tfe_ref, out_ref, attn_ref):
    sim = sim_ref[...]
    m = jnp.max(sim, axis=-1, keepdims=True)
    e = jnp.exp(sim - m)
    z = jnp.sum(e, axis=-1, keepdims=True)

    def step(_, t):
        return jnp.max(jnp.where(e < t, e, -1.0), axis=-1, keepdims=True)

    t = lax.fori_loop(0, _TOPK, step, jnp.full_like(z, jnp.inf))
    masked = jnp.where(e >= t, e, 0.0)
    s = jnp.sum(masked, axis=-1, keepdims=True)
    attn = masked * (1.0 / (s + 1e-8 * z))
    attn_ref[...] = attn
    out = jax.lax.dot_general(
        tfe_ref[...], attn.astype(jnp.bfloat16), (((1,), (1,)), ((), ())),
        preferred_element_type=jnp.float32,
    )
    out_ref[...] = scale_ref[0, 0] * out


def _b_body(scale_ref, sim_ref, thr_ref, tfe_ref, out_ref, attn_ref):
    sim = sim_ref[...]
    t = thr_ref[...][:, 0:1]
    m = jnp.max(sim, axis=-1, keepdims=True)
    e = jnp.exp(sim - m)
    z = jnp.sum(e, axis=-1, keepdims=True)
    masked = jnp.where(sim >= t, e, 0.0)
    s = jnp.sum(masked, axis=-1, keepdims=True)
    attn = masked * (1.0 / (s + 1e-8 * z))
    attn_ref[...] = attn
    out = jax.lax.dot_general(
        tfe_ref[...], attn.astype(jnp.bfloat16), (((1,), (1,)), ((), ())),
        preferred_element_type=jnp.float32,
    )
    out_ref[...] = scale_ref[0, 0] * out


def kernel(tg_dec, tf_base, tf_expr, scale):
    g, d = tg_dec.shape
    t_dim = tf_base.shape[0]
    p = tf_expr.shape[0]
    r = 512
    grid = ((g + r - 1) // r,)
    nb1 = 0
    g1 = 0
    scale2 = jnp.asarray(scale, jnp.float32).reshape(1, 1)
    tfe_bf = tf_expr.astype(jnp.bfloat16)

    sim = pl.pallas_call(
        functools.partial(_a_body, d=d),
        grid=grid,
        in_specs=[
            pl.BlockSpec((r, d), lambda i: (i, 0)),
            pl.BlockSpec((t_dim, d), lambda i: (0, 0)),
        ],
        out_specs=pl.BlockSpec((r, t_dim), lambda i: (i, 0)),
        out_shape=jax.ShapeDtypeStruct((g, t_dim), jnp.float32),
        compiler_params=pltpu.CompilerParams(
            dimension_semantics=("parallel",),
        ),
    )(tg_dec, tf_base)

    mesh = plsc.VectorSubcoreMesh(core_axis_name="c", subcore_axis_name="s")
    thr = pl.kernel(
        functools.partial(_sc_body, t_dim=t_dim, j0=0,
                          n_batches=g // _RB),
        out_type=jax.ShapeDtypeStruct((g, _L), jnp.float32),
        mesh=mesh,
        compiler_params=pltpu.CompilerParams(needs_layout_passes=False),
        scratch_types=[
            pltpu.VMEM((2 * _RB, t_dim), jnp.float32),
            pltpu.VMEM((2 * _CAND,), jnp.float32),
            pltpu.VMEM((_RB, _L), jnp.float32),
            pltpu.SemaphoreType.DMA,
        ],
    )(sim)

    tf_scalar, attn = pl.pallas_call(
        _b_body,
        grid=grid,
        in_specs=[
            pl.BlockSpec((1, 1), lambda i: (0, 0), memory_space=pltpu.SMEM),
            pl.BlockSpec((r, t_dim), lambda i: (i, 0)),
            pl.BlockSpec((r, _L), lambda i: (i, 0)),
            pl.BlockSpec((p, t_dim), lambda i: (0, 0)),
        ],
        out_specs=(
            pl.BlockSpec((p, r), lambda i: (0, i)),
            pl.BlockSpec((r, t_dim), lambda i: (i, 0)),
        ),
        out_shape=(
            jax.ShapeDtypeStruct((p, g), jnp.float32),
            jax.ShapeDtypeStruct((g, t_dim), jnp.float32),
        ),
        compiler_params=pltpu.CompilerParams(
            dimension_semantics=("parallel",),
        ),
    )(scale2, sim, thr, tfe_bf)
    return tf_scalar, attn
